# baseline (speedup 1.0000x reference)
import functools

import jax
import jax.numpy as jnp
from jax import lax
from jax.experimental import pallas as pl
from jax.experimental.pallas import tpu as pltpu
from jax.experimental.pallas import tpu_sc as plsc

D_MODEL = 1024
E = 8
CAP = 1024
T = 8192
HIDDEN = 2730
TB = 1024
NT = T // TB
PAD_ROWS = 1024
XROWS = E * CAP + PAD_ROWS
NC, NS, L = 2, 16, 16
NW = NC * NS


def _router_body(x_ref, gw_ref, dest0_ref, dest1_ref, w0_ref, w1_ref,
                 p1_s, p2_s, i1_s, i2_s):
    i = pl.program_id(0)
    xb = x_ref[...]
    gw = gw_ref[...]
    logits = lax.dot_general(xb, gw, (((1,), (1,)), ((), ())),
                             preferred_element_type=jnp.float32)
    cols = lax.broadcasted_iota(jnp.int32, (TB, E), 1)
    m1 = jnp.max(logits, axis=1, keepdims=True)
    i1 = jnp.min(jnp.where(logits == m1, cols, E), axis=1)
    masked = jnp.where(cols == i1[:, None], -jnp.inf, logits)
    m2 = jnp.max(masked, axis=1, keepdims=True)
    i2 = jnp.min(jnp.where(masked == m2, cols, E), axis=1)
    e2 = jnp.exp(m2 - m1)[:, 0]
    s = 1.0 + e2
    p1 = 1.0 / s
    p2 = e2 / s
    s2 = p1 + p2
    q1 = p1 / s2
    q2 = p2 / s2
    p1_s[i, :] = q1
    p2_s[i, :] = q2
    i1_s[i, :] = i1
    i2_s[i, :] = i2

    @pl.when(i == NT - 1)
    def _select():
        qA = p1_s[...]
        qB = p2_s[...]
        eA = i1_s[...]
        eB = i2_s[...]
        bA = lax.bitcast_convert_type(qA, jnp.int32)
        bB = lax.bitcast_convert_type(qB, jnp.int32)

        capf = jnp.float32(CAP)

        def bs_body(_, carry):
            lo, hi = carry
            nlo, nhi = [], []
            for e in range(E):
                mid = (lo[e] + hi[e]) >> 1
                cnt = (jnp.sum(((eA == e) & (bA > mid)).astype(jnp.float32)) +
                       jnp.sum(((eB == e) & (bB > mid)).astype(jnp.float32)))
                lt = cnt < capf
                nlo.append(jnp.where(lt, lo[e], mid))
                nhi.append(jnp.where(lt, mid, hi[e]))
            return tuple(nlo), tuple(nhi)

        lo0 = tuple(jnp.int32(-1) for _ in range(E))
        hi0 = tuple(jnp.int32(0x3F800001) for _ in range(E))
        _, vstar = lax.fori_loop(0, 31, bs_body, (lo0, hi0))

        rem = []
        for e in range(E):
            cnt_gt = (jnp.sum(((eA == e) & (bA > vstar[e])).astype(jnp.float32)) +
                      jnp.sum(((eB == e) & (bB > vstar[e])).astype(jnp.float32)))
            rem.append(capf - cnt_gt)

        def sel(eplane, per_e):
            acc = jnp.zeros((NT, TB), per_e[0].dtype)
            for e in range(E):
                acc = jnp.where(eplane == e, per_e[e], acc)
            return acc

        vsA = sel(eA, [jnp.broadcast_to(v, (NT, TB)) for v in vstar])
        vsB = sel(eB, [jnp.broadcast_to(v, (NT, TB)) for v in vstar])
        gtA, eqA = bA > vsA, bA == vsA
        gtB, eqB = bB > vsB, bB == vsB

        U = (lax.broadcasted_iota(jnp.int32, (TB, TB), 0) <
             lax.broadcasted_iota(jnp.int32, (TB, TB), 1)).astype(jnp.float32)
        rI = lax.broadcasted_iota(jnp.int32, (2 * E * NT, 2 * E * NT), 0)
        rJ = lax.broadcasted_iota(jnp.int32, (2 * E * NT, 2 * E * NT), 1)
        Loff = (((rI // NT) == (rJ // NT)) & (rJ < rI)).astype(jnp.float32)

        def prefix_planes(planesA, planesB):
            Z = jnp.concatenate(planesA + planesB, axis=0)
            P = lax.dot_general(Z, U, (((1,), (0,)), ((), ())),
                                preferred_element_type=jnp.float32)
            rs = P[:, TB - 1:TB] + Z[:, TB - 1:TB]
            offs = lax.dot_general(Loff, rs, (((1,), (0,)), ((), ())),
                                   preferred_element_type=jnp.float32)
            P = P + offs
            pA = [P[e * NT:(e + 1) * NT, :] for e in range(E)]
            pB = [P[(E + e) * NT:(E + e + 1) * NT, :] for e in range(E)]
            return pA, pB

        eqA_e = [((eA == e) & eqA).astype(jnp.float32) for e in range(E)]
        eqB_e = [((eB == e) & eqB).astype(jnp.float32) for e in range(E)]
        tpA, tpB = prefix_planes(eqA_e, eqB_e)
        tieA = sel(eA, [tpA[e] + tpB[e] for e in range(E)])
        tieB = sel(eB, [tpA[e] + eqA_e[e] + tpB[e] for e in range(E)])
        remA = sel(eA, [jnp.broadcast_to(r, (NT, TB)) for r in rem])
        remB = sel(eB, [jnp.broadcast_to(r, (NT, TB)) for r in rem])
        keepA = gtA | (eqA & (tieA < remA))
        keepB = gtB | (eqB & (tieB < remB))

        kA_e = [((eA == e) & keepA).astype(jnp.float32) for e in range(E)]
        kB_e = [((eB == e) & keepB).astype(jnp.float32) for e in range(E)]
        ppA, ppB = prefix_planes(kA_e, kB_e)
        posA = sel(eA, [ppA[e] + ppB[e] for e in range(E)])
        posB = sel(eB, [ppA[e] + kA_e[e] + ppB[e] for e in range(E)])

        destA = jnp.where(keepA, eA * CAP + posA.astype(jnp.int32),
                          jnp.int32(E * CAP))
        destB = jnp.where(keepB, eB * CAP + posB.astype(jnp.int32),
                          jnp.int32(E * CAP))
        dest0_ref[...] = destA
        dest1_ref[...] = destB
        w0_ref[...] = jnp.where(keepA, qA, 0.0)
        w1_ref[...] = jnp.where(keepB, qB, 0.0)


def _router(x, gate_w):
    return pl.pallas_call(
        _router_body,
        grid=(NT,),
        in_specs=[
            pl.BlockSpec((TB, D_MODEL), lambda i: (i, 0)),
            pl.BlockSpec((E, D_MODEL), lambda i: (0, 0)),
        ],
        out_specs=[
            pl.BlockSpec((NT, TB), lambda i: (0, 0)),
            pl.BlockSpec((NT, TB), lambda i: (0, 0)),
            pl.BlockSpec((NT, TB), lambda i: (0, 0)),
            pl.BlockSpec((NT, TB), lambda i: (0, 0)),
        ],
        out_shape=[
            jax.ShapeDtypeStruct((NT, TB), jnp.int32),
            jax.ShapeDtypeStruct((NT, TB), jnp.int32),
            jax.ShapeDtypeStruct((NT, TB), jnp.float32),
            jax.ShapeDtypeStruct((NT, TB), jnp.float32),
        ],
        scratch_shapes=[
            pltpu.VMEM((NT, TB), jnp.float32),
            pltpu.VMEM((NT, TB), jnp.float32),
            pltpu.VMEM((NT, TB), jnp.int32),
            pltpu.VMEM((NT, TB), jnp.int32),
        ],
        compiler_params=pltpu.CompilerParams(
            dimension_semantics=("arbitrary",)),
    )(x, gate_w)


_SC_MESH = plsc.VectorSubcoreMesh(core_axis_name="c", subcore_axis_name="s")
_SC_PARAMS = pltpu.CompilerParams(needs_layout_passes=False)


def _invert_body(d0_hbm, d1_hbm, w0_hbm, w1_hbm, src_hbm, wts_hbm,
                 d_v, wv_v, src_v, wts_v):
    wid = lax.axis_index("s") * NC + lax.axis_index("c")

    @pl.when(wid == 0)
    def _():
        zi = jnp.zeros((L,), jnp.int32)
        zf = jnp.zeros((L,), jnp.float32)

        def zero_body(i, _):
            src_v[pl.ds(i * L, L)] = zi
            wts_v[pl.ds(i * L, L)] = zf
            return 0

        lax.fori_loop(0, XROWS // L, zero_body, 0)

        lane = lax.iota(jnp.int32, L)

        def scan_plane(d_hbm, w_hbm):
            pltpu.sync_copy(d_hbm, d_v)
            pltpu.sync_copy(w_hbm, wv_v)

            def body(i, _):
                d = d_v[pl.ds(i * L, L)]
                w = wv_v[pl.ds(i * L, L)]
                tok = i * L + lane
                plsc.store_scatter(src_v, [d], tok)
                plsc.store_scatter(wts_v, [d], w)
                return 0

            lax.fori_loop(0, T // L, body, 0)

        scan_plane(d0_hbm, w0_hbm)
        scan_plane(d1_hbm, w1_hbm)
        pltpu.sync_copy(src_v, src_hbm)
        pltpu.sync_copy(wts_v, wts_hbm)


@functools.partial(
    pl.kernel,
    out_type=[jax.ShapeDtypeStruct((XROWS,), jnp.int32),
              jax.ShapeDtypeStruct((XROWS,), jnp.float32)],
    mesh=_SC_MESH,
    scratch_types=[pltpu.VMEM((T,), jnp.int32),
                   pltpu.VMEM((T,), jnp.float32),
                   pltpu.VMEM((XROWS,), jnp.int32),
                   pltpu.VMEM((XROWS,), jnp.float32)],
    compiler_params=_SC_PARAMS,
)
def _invert(*args):
    _invert_body(*args)


RPW = XROWS // NW
GCH = 48
NCH = RPW // GCH


def _gather_body(x_hbm, src_hbm, xe_hbm, idx_v, buf, sem):
    wid = lax.axis_index("s") * NC + lax.axis_index("c")
    pltpu.sync_copy(src_hbm.at[wid], idx_v)
    base = wid * RPW
    for j in range(NCH):
        pltpu.async_copy(x_hbm.at[idx_v.at[j]], buf, sem).wait()
        pltpu.sync_copy(buf, xe_hbm.at[pl.ds(base + j * GCH, GCH)])


@functools.partial(
    pl.kernel,
    out_type=jax.ShapeDtypeStruct((XROWS, D_MODEL), jnp.float32),
    mesh=_SC_MESH,
    scratch_types=[pltpu.VMEM((NCH, GCH), jnp.int32),
                   pltpu.VMEM((GCH, D_MODEL), jnp.float32),
                   pltpu.SemaphoreType.DMA],
    compiler_params=_SC_PARAMS,
)
def _gather(*args):
    _gather_body(*args)


HCH = 1024
NH = (HIDDEN + HCH - 1) // HCH
NE = XROWS // CAP


def _ffn_body(xe_ref, wg_ref, wu_ref, wd_ref, wts_ref, ye_ref, yacc):
    j = pl.program_id(1)
    xb = xe_ref[...].astype(jnp.bfloat16)
    wg = wg_ref[0].astype(jnp.bfloat16)
    wu = wu_ref[0].astype(jnp.bfloat16)
    wd = wd_ref[0]
    g = lax.dot_general(xb, wg, (((1,), (1,)), ((), ())),
                        preferred_element_type=jnp.float32)
    u = lax.dot_general(xb, wu, (((1,), (1,)), ((), ())),
                        preferred_element_type=jnp.float32)
    hid = lax.broadcasted_iota(jnp.int32, (CAP, HCH), 1) + j * HCH
    hmask = hid < HIDDEN
    h = jnp.where(hmask, g * jax.nn.sigmoid(g) * u, 0.0).astype(jnp.bfloat16)
    wdm = jnp.where(lax.broadcasted_iota(jnp.int32, (D_MODEL, HCH), 1) + j * HCH
                    < HIDDEN, wd, 0.0).astype(jnp.bfloat16)
    y = lax.dot_general(h, wdm, (((1,), (1,)), ((), ())),
                        preferred_element_type=jnp.float32)

    @pl.when(j == 0)
    def _():
        yacc[...] = y

    @pl.when(j > 0)
    def _():
        yacc[...] = yacc[...] + y

    @pl.when(j == NH - 1)
    def _():
        ye_ref[...] = yacc[...] * wts_ref[0]


def _ffn(xe, w_gate, w_up, w_down, wts3):
    return pl.pallas_call(
        _ffn_body,
        grid=(NE, NH),
        in_specs=[
            pl.BlockSpec((CAP, D_MODEL), lambda e, j: (e, 0)),
            pl.BlockSpec((1, HCH, D_MODEL),
                         lambda e, j: (jnp.minimum(e, E - 1), j, 0)),
            pl.BlockSpec((1, HCH, D_MODEL),
                         lambda e, j: (jnp.minimum(e, E - 1), j, 0)),
            pl.BlockSpec((1, D_MODEL, HCH),
                         lambda e, j: (jnp.minimum(e, E - 1), 0, j)),
            pl.BlockSpec((1, CAP, 1), lambda e, j: (e, 0, 0)),
        ],
        out_specs=pl.BlockSpec((CAP, D_MODEL), lambda e, j: (e, 0)),
        out_shape=jax.ShapeDtypeStruct((XROWS, D_MODEL), jnp.float32),
        scratch_shapes=[pltpu.VMEM((CAP, D_MODEL), jnp.float32)],
        compiler_params=pltpu.CompilerParams(
            dimension_semantics=("arbitrary", "arbitrary")),
    )(xe, w_gate, w_up, w_down, wts3)


TPW = T // NW
CCH = 32
NCC = TPW // CCH


def _combine_body(ye_hbm, d0_hbm, d1_hbm, out_hbm, i0_v, i1_v, b0, b1, ob, sem):
    wid = lax.axis_index("s") * NC + lax.axis_index("c")
    wbase = wid * TPW
    for cc in range(NCC):
        base = wbase + cc * CCH
        pltpu.sync_copy(d0_hbm.at[pl.ds(base, CCH)], i0_v)
        pltpu.sync_copy(d1_hbm.at[pl.ds(base, CCH)], i1_v)
        pltpu.async_copy(ye_hbm.at[i0_v], b0, sem).wait()
        pltpu.async_copy(ye_hbm.at[i1_v], b1, sem).wait()

        def row_body(r, _):
            for c in range(D_MODEL // L):
                ob[r, pl.ds(c * L, L)] = (b0[r, pl.ds(c * L, L)] +
                                          b1[r, pl.ds(c * L, L)])
            return 0

        lax.fori_loop(0, CCH, row_body, 0)
        pltpu.sync_copy(ob, out_hbm.at[pl.ds(base, CCH)])


@functools.partial(
    pl.kernel,
    out_type=jax.ShapeDtypeStruct((T, D_MODEL), jnp.float32),
    mesh=_SC_MESH,
    scratch_types=[pltpu.VMEM((CCH,), jnp.int32),
                   pltpu.VMEM((CCH,), jnp.int32),
                   pltpu.VMEM((CCH, D_MODEL), jnp.float32),
                   pltpu.VMEM((CCH, D_MODEL), jnp.float32),
                   pltpu.VMEM((CCH, D_MODEL), jnp.float32),
                   pltpu.SemaphoreType.DMA],
    compiler_params=_SC_PARAMS,
)
def _combine(*args):
    _combine_body(*args)


def kernel(x, gate_w, w_gate, w_up, w_down):
    dest0, dest1, w0, w1 = _router(x, gate_w)
    src, wts = _invert(dest0.reshape(T), dest1.reshape(T),
                       w0.reshape(T), w1.reshape(T))
    xe = _gather(x, src.reshape(NW, NCH, GCH))
    ye = _ffn(xe, w_gate, w_up, w_down, wts.reshape(NE, CAP, 1))
    out = _combine(ye, dest0.reshape(T), dest1.reshape(T))
    return out


# speedup vs baseline: 1.0005x; 1.0005x over previous
import functools

import jax
import jax.numpy as jnp
from jax import lax
from jax.experimental import pallas as pl
from jax.experimental.pallas import tpu as pltpu
from jax.experimental.pallas import tpu_sc as plsc

D_MODEL = 1024
E = 8
CAP = 1024
T = 8192
HIDDEN = 2730
TB = 1024
NT = T // TB
PAD_ROWS = 1024
XROWS = E * CAP + PAD_ROWS
NC, NS, L = 2, 16, 16
NW = NC * NS


def _router_body(x_ref, gw_ref, dest0_ref, dest1_ref, w0_ref, w1_ref,
                 p1_s, p2_s, i1_s, i2_s):
    i = pl.program_id(0)
    xb = x_ref[...]
    gw = gw_ref[...]
    logits = lax.dot_general(xb, gw, (((1,), (1,)), ((), ())),
                             preferred_element_type=jnp.float32)
    cols = lax.broadcasted_iota(jnp.int32, (TB, E), 1)
    m1 = jnp.max(logits, axis=1, keepdims=True)
    i1 = jnp.min(jnp.where(logits == m1, cols, E), axis=1)
    masked = jnp.where(cols == i1[:, None], -jnp.inf, logits)
    m2 = jnp.max(masked, axis=1, keepdims=True)
    i2 = jnp.min(jnp.where(masked == m2, cols, E), axis=1)
    e2 = jnp.exp(m2 - m1)[:, 0]
    s = 1.0 + e2
    p1 = 1.0 / s
    p2 = e2 / s
    s2 = p1 + p2
    q1 = p1 / s2
    q2 = p2 / s2
    p1_s[i, :] = q1
    p2_s[i, :] = q2
    i1_s[i, :] = i1
    i2_s[i, :] = i2

    @pl.when(i == NT - 1)
    def _select():
        qA = p1_s[...]
        qB = p2_s[...]
        eA = i1_s[...]
        eB = i2_s[...]
        bA = lax.bitcast_convert_type(qA, jnp.int32)
        bB = lax.bitcast_convert_type(qB, jnp.int32)

        capf = jnp.float32(CAP)

        def bs_body(_, carry):
            lo, hi = carry
            nlo, nhi = [], []
            for e in range(E):
                mid = (lo[e] + hi[e]) >> 1
                cnt = (jnp.sum(((eA == e) & (bA > mid)).astype(jnp.float32)) +
                       jnp.sum(((eB == e) & (bB > mid)).astype(jnp.float32)))
                lt = cnt < capf
                nlo.append(jnp.where(lt, lo[e], mid))
                nhi.append(jnp.where(lt, mid, hi[e]))
            return tuple(nlo), tuple(nhi)

        lo0 = tuple(jnp.int32(-1) for _ in range(E))
        hi0 = tuple(jnp.int32(0x3F800001) for _ in range(E))
        _, vstar = lax.fori_loop(0, 31, bs_body, (lo0, hi0))

        rem = []
        for e in range(E):
            cnt_gt = (jnp.sum(((eA == e) & (bA > vstar[e])).astype(jnp.float32)) +
                      jnp.sum(((eB == e) & (bB > vstar[e])).astype(jnp.float32)))
            rem.append(capf - cnt_gt)

        def sel(eplane, per_e):
            acc = jnp.zeros((NT, TB), per_e[0].dtype)
            for e in range(E):
                acc = jnp.where(eplane == e, per_e[e], acc)
            return acc

        vsA = sel(eA, [jnp.broadcast_to(v, (NT, TB)) for v in vstar])
        vsB = sel(eB, [jnp.broadcast_to(v, (NT, TB)) for v in vstar])
        gtA, eqA = bA > vsA, bA == vsA
        gtB, eqB = bB > vsB, bB == vsB

        U = (lax.broadcasted_iota(jnp.int32, (TB, TB), 0) <
             lax.broadcasted_iota(jnp.int32, (TB, TB), 1)).astype(jnp.float32)
        rI = lax.broadcasted_iota(jnp.int32, (2 * E * NT, 2 * E * NT), 0)
        rJ = lax.broadcasted_iota(jnp.int32, (2 * E * NT, 2 * E * NT), 1)
        Loff = (((rI // NT) == (rJ // NT)) & (rJ < rI)).astype(jnp.float32)

        def prefix_planes(planesA, planesB):
            Z = jnp.concatenate(planesA + planesB, axis=0)
            P = lax.dot_general(Z, U, (((1,), (0,)), ((), ())),
                                preferred_element_type=jnp.float32)
            rs = P[:, TB - 1:TB] + Z[:, TB - 1:TB]
            offs = lax.dot_general(Loff, rs, (((1,), (0,)), ((), ())),
                                   preferred_element_type=jnp.float32)
            P = P + offs
            pA = [P[e * NT:(e + 1) * NT, :] for e in range(E)]
            pB = [P[(E + e) * NT:(E + e + 1) * NT, :] for e in range(E)]
            return pA, pB

        eqA_e = [((eA == e) & eqA).astype(jnp.float32) for e in range(E)]
        eqB_e = [((eB == e) & eqB).astype(jnp.float32) for e in range(E)]
        tpA, tpB = prefix_planes(eqA_e, eqB_e)
        tieA = sel(eA, [tpA[e] + tpB[e] for e in range(E)])
        tieB = sel(eB, [tpA[e] + eqA_e[e] + tpB[e] for e in range(E)])
        remA = sel(eA, [jnp.broadcast_to(r, (NT, TB)) for r in rem])
        remB = sel(eB, [jnp.broadcast_to(r, (NT, TB)) for r in rem])
        keepA = gtA | (eqA & (tieA < remA))
        keepB = gtB | (eqB & (tieB < remB))

        kA_e = [((eA == e) & keepA).astype(jnp.float32) for e in range(E)]
        kB_e = [((eB == e) & keepB).astype(jnp.float32) for e in range(E)]
        ppA, ppB = prefix_planes(kA_e, kB_e)
        posA = sel(eA, [ppA[e] + ppB[e] for e in range(E)])
        posB = sel(eB, [ppA[e] + kA_e[e] + ppB[e] for e in range(E)])

        destA = jnp.where(keepA, eA * CAP + posA.astype(jnp.int32),
                          jnp.int32(E * CAP))
        destB = jnp.where(keepB, eB * CAP + posB.astype(jnp.int32),
                          jnp.int32(E * CAP))
        dest0_ref[...] = destA
        dest1_ref[...] = destB
        w0_ref[...] = jnp.where(keepA, qA, 0.0)
        w1_ref[...] = jnp.where(keepB, qB, 0.0)


def _router(x, gate_w):
    return pl.pallas_call(
        _router_body,
        grid=(NT,),
        in_specs=[
            pl.BlockSpec((TB, D_MODEL), lambda i: (i, 0)),
            pl.BlockSpec((E, D_MODEL), lambda i: (0, 0)),
        ],
        out_specs=[
            pl.BlockSpec((NT, TB), lambda i: (0, 0)),
            pl.BlockSpec((NT, TB), lambda i: (0, 0)),
            pl.BlockSpec((NT, TB), lambda i: (0, 0)),
            pl.BlockSpec((NT, TB), lambda i: (0, 0)),
        ],
        out_shape=[
            jax.ShapeDtypeStruct((NT, TB), jnp.int32),
            jax.ShapeDtypeStruct((NT, TB), jnp.int32),
            jax.ShapeDtypeStruct((NT, TB), jnp.float32),
            jax.ShapeDtypeStruct((NT, TB), jnp.float32),
        ],
        scratch_shapes=[
            pltpu.VMEM((NT, TB), jnp.float32),
            pltpu.VMEM((NT, TB), jnp.float32),
            pltpu.VMEM((NT, TB), jnp.int32),
            pltpu.VMEM((NT, TB), jnp.int32),
        ],
        compiler_params=pltpu.CompilerParams(
            dimension_semantics=("arbitrary",)),
    )(x, gate_w)


_SC_MESH = plsc.VectorSubcoreMesh(core_axis_name="c", subcore_axis_name="s")
_SC_PARAMS = pltpu.CompilerParams(needs_layout_passes=False)


def _invert_body(d0_hbm, d1_hbm, w0_hbm, w1_hbm, src_hbm, wts_hbm,
                 d_v, wv_v, src_v, wts_v):
    wid = lax.axis_index("s") * NC + lax.axis_index("c")

    @pl.when(wid == 0)
    def _():
        zi = jnp.zeros((L,), jnp.int32)
        zf = jnp.zeros((L,), jnp.float32)

        def zero_body(i, _):
            src_v[pl.ds(i * L, L)] = zi
            wts_v[pl.ds(i * L, L)] = zf
            return 0

        lax.fori_loop(0, XROWS // L, zero_body, 0)

        lane = lax.iota(jnp.int32, L)

        def scan_plane(d_hbm, w_hbm):
            pltpu.sync_copy(d_hbm, d_v)
            pltpu.sync_copy(w_hbm, wv_v)

            def body(i, _):
                d = d_v[pl.ds(i * L, L)]
                w = wv_v[pl.ds(i * L, L)]
                tok = i * L + lane
                plsc.store_scatter(src_v, [d], tok)
                plsc.store_scatter(wts_v, [d], w)
                return 0

            lax.fori_loop(0, T // L, body, 0)

        scan_plane(d0_hbm, w0_hbm)
        scan_plane(d1_hbm, w1_hbm)
        pltpu.sync_copy(src_v, src_hbm)
        pltpu.sync_copy(wts_v, wts_hbm)


@functools.partial(
    pl.kernel,
    out_type=[jax.ShapeDtypeStruct((XROWS,), jnp.int32),
              jax.ShapeDtypeStruct((XROWS,), jnp.float32)],
    mesh=_SC_MESH,
    scratch_types=[pltpu.VMEM((T,), jnp.int32),
                   pltpu.VMEM((T,), jnp.float32),
                   pltpu.VMEM((XROWS,), jnp.int32),
                   pltpu.VMEM((XROWS,), jnp.float32)],
    compiler_params=_SC_PARAMS,
)
def _invert(*args):
    _invert_body(*args)


RPW = XROWS // NW
GCH = 48
NCH = RPW // GCH


def _gather_body(x_hbm, src_hbm, xe_hbm, idx_v, bufa, bufb, sema, semb):
    wid = lax.axis_index("s") * NC + lax.axis_index("c")
    pltpu.sync_copy(src_hbm.at[wid], idx_v)
    base = wid * RPW
    bufs = (bufa, bufb)
    sems = (sema, semb)
    pltpu.async_copy(x_hbm.at[idx_v.at[0]], bufa, sema)
    for j in range(NCH):
        pltpu.make_async_copy(x_hbm.at[idx_v.at[j]], bufs[j % 2],
                              sems[j % 2]).wait()
        if j + 1 < NCH:
            pltpu.async_copy(x_hbm.at[idx_v.at[j + 1]], bufs[(j + 1) % 2],
                             sems[(j + 1) % 2])
        pltpu.sync_copy(bufs[j % 2], xe_hbm.at[pl.ds(base + j * GCH, GCH)])


@functools.partial(
    pl.kernel,
    out_type=jax.ShapeDtypeStruct((XROWS, D_MODEL), jnp.float32),
    mesh=_SC_MESH,
    scratch_types=[pltpu.VMEM((NCH, GCH), jnp.int32),
                   pltpu.VMEM((GCH, D_MODEL), jnp.float32),
                   pltpu.VMEM((GCH, D_MODEL), jnp.float32),
                   pltpu.SemaphoreType.DMA,
                   pltpu.SemaphoreType.DMA],
    compiler_params=_SC_PARAMS,
)
def _gather(*args):
    _gather_body(*args)


HCH = 1024
NH = (HIDDEN + HCH - 1) // HCH
NE = XROWS // CAP


def _ffn_body(xe_ref, wg_ref, wu_ref, wd_ref, wts_ref, ye_ref, yacc):
    e = pl.program_id(0)
    j = pl.program_id(1)

    @pl.when(jnp.logical_and(e == NE - 1, j == NH - 1))
    def _():
        ye_ref[...] = jnp.zeros((CAP, D_MODEL), jnp.float32)

    @pl.when(e < NE - 1)
    def _():
        _ffn_compute(xe_ref, wg_ref, wu_ref, wd_ref, wts_ref, ye_ref, yacc, j)


def _ffn_compute(xe_ref, wg_ref, wu_ref, wd_ref, wts_ref, ye_ref, yacc, j):
    xb = xe_ref[...].astype(jnp.bfloat16)
    wg = wg_ref[0].astype(jnp.bfloat16)
    wu = wu_ref[0].astype(jnp.bfloat16)
    wd = wd_ref[0]
    g = lax.dot_general(xb, wg, (((1,), (1,)), ((), ())),
                        preferred_element_type=jnp.float32)
    u = lax.dot_general(xb, wu, (((1,), (1,)), ((), ())),
                        preferred_element_type=jnp.float32)
    hid = lax.broadcasted_iota(jnp.int32, (CAP, HCH), 1) + j * HCH
    hmask = hid < HIDDEN
    h = jnp.where(hmask, g * jax.nn.sigmoid(g) * u, 0.0).astype(jnp.bfloat16)
    wdm = jnp.where(lax.broadcasted_iota(jnp.int32, (D_MODEL, HCH), 1) + j * HCH
                    < HIDDEN, wd, 0.0).astype(jnp.bfloat16)
    y = lax.dot_general(h, wdm, (((1,), (1,)), ((), ())),
                        preferred_element_type=jnp.float32)

    @pl.when(j == 0)
    def _():
        yacc[...] = y

    @pl.when(j > 0)
    def _():
        yacc[...] = yacc[...] + y

    @pl.when(j == NH - 1)
    def _():
        ye_ref[...] = yacc[...] * wts_ref[0]


def _ffn(xe, w_gate, w_up, w_down, wts3):
    return pl.pallas_call(
        _ffn_body,
        grid=(NE, NH),
        in_specs=[
            pl.BlockSpec((CAP, D_MODEL), lambda e, j: (e, 0)),
            pl.BlockSpec((1, HCH, D_MODEL),
                         lambda e, j: (jnp.minimum(e, E - 1), j, 0)),
            pl.BlockSpec((1, HCH, D_MODEL),
                         lambda e, j: (jnp.minimum(e, E - 1), j, 0)),
            pl.BlockSpec((1, D_MODEL, HCH),
                         lambda e, j: (jnp.minimum(e, E - 1), 0, j)),
            pl.BlockSpec((1, CAP, 1), lambda e, j: (e, 0, 0)),
        ],
        out_specs=pl.BlockSpec((CAP, D_MODEL), lambda e, j: (e, 0)),
        out_shape=jax.ShapeDtypeStruct((XROWS, D_MODEL), jnp.float32),
        scratch_shapes=[pltpu.VMEM((CAP, D_MODEL), jnp.float32)],
        compiler_params=pltpu.CompilerParams(
            dimension_semantics=("arbitrary", "arbitrary")),
    )(xe, w_gate, w_up, w_down, wts3)


TPW = T // NW
CCH = 16
NCC = TPW // CCH


def _combine_body(ye_hbm, d0_hbm, d1_hbm, out_hbm, i0_v, i1_v,
                  b0a, b1a, b0b, b1b, oba, obb, sema, semb, semoa, semob):
    wid = lax.axis_index("s") * NC + lax.axis_index("c")
    wbase = wid * TPW
    pltpu.sync_copy(d0_hbm.at[wid], i0_v)
    pltpu.sync_copy(d1_hbm.at[wid], i1_v)
    b0s, b1s, obs = (b0a, b0b), (b1a, b1b), (oba, obb)
    sems = (sema, semb)
    semos = (semoa, semob)

    def issue(c, p):
        pltpu.async_copy(ye_hbm.at[i0_v.at[c]], b0s[p], sems[p])
        pltpu.async_copy(ye_hbm.at[i1_v.at[c]], b1s[p], sems[p])

    def wait(c, p):
        pltpu.make_async_copy(ye_hbm.at[i0_v.at[c]], b0s[p], sems[p]).wait()
        pltpu.make_async_copy(ye_hbm.at[i1_v.at[c]], b1s[p], sems[p]).wait()

    def out_wait(p):
        pltpu.make_async_copy(obs[p], out_hbm.at[pl.ds(wbase, CCH)],
                              semos[p]).wait()

    def compute_store(c, p):
        b0, b1, ob = b0s[p], b1s[p], obs[p]

        @pl.when(c >= 2)
        def _():
            out_wait(p)

        def row_body(r, _):
            for k in range(D_MODEL // L):
                ob[r, pl.ds(k * L, L)] = (b0[r, pl.ds(k * L, L)] +
                                          b1[r, pl.ds(k * L, L)])
            return 0

        lax.fori_loop(0, CCH, row_body, 0)
        pltpu.async_copy(ob, out_hbm.at[pl.ds(wbase + c * CCH, CCH)], semos[p])

    issue(0, 0)

    def step(s, _):
        c0 = s * 2
        wait(c0, 0)
        issue(c0 + 1, 1)
        compute_store(c0, 0)
        wait(c0 + 1, 1)

        @pl.when(s < NCC // 2 - 1)
        def _():
            issue(c0 + 2, 0)

        compute_store(c0 + 1, 1)
        return 0

    lax.fori_loop(0, NCC // 2, step, 0)
    out_wait(0)
    out_wait(1)


@functools.partial(
    pl.kernel,
    out_type=jax.ShapeDtypeStruct((T, D_MODEL), jnp.float32),
    mesh=_SC_MESH,
    scratch_types=[pltpu.VMEM((NCC, CCH), jnp.int32),
                   pltpu.VMEM((NCC, CCH), jnp.int32),
                   pltpu.VMEM((CCH, D_MODEL), jnp.float32),
                   pltpu.VMEM((CCH, D_MODEL), jnp.float32),
                   pltpu.VMEM((CCH, D_MODEL), jnp.float32),
                   pltpu.VMEM((CCH, D_MODEL), jnp.float32),
                   pltpu.VMEM((CCH, D_MODEL), jnp.float32),
                   pltpu.VMEM((CCH, D_MODEL), jnp.float32),
                   pltpu.SemaphoreType.DMA,
                   pltpu.SemaphoreType.DMA,
                   pltpu.SemaphoreType.DMA,
                   pltpu.SemaphoreType.DMA],
    compiler_params=_SC_PARAMS,
)
def _combine(*args):
    _combine_body(*args)


def kernel(x, gate_w, w_gate, w_up, w_down):
    dest0, dest1, w0, w1 = _router(x, gate_w)
    src, wts = _invert(dest0.reshape(T), dest1.reshape(T),
                       w0.reshape(T), w1.reshape(T))
    xe = _gather(x, src.reshape(NW, NCH, GCH))
    ye = _ffn(xe, w_gate, w_up, w_down, wts.reshape(NE, CAP, 1))
    out = _combine(ye, dest0.reshape(NW, NCC, CCH), dest1.reshape(NW, NCC, CCH))
    return out


# speedup vs baseline: 1.0379x; 1.0375x over previous
import functools

import jax
import jax.numpy as jnp
from jax import lax
from jax.experimental import pallas as pl
from jax.experimental.pallas import tpu as pltpu
from jax.experimental.pallas import tpu_sc as plsc

D_MODEL = 1024
E = 8
CAP = 1024
T = 8192
HIDDEN = 2730
TB = 1024
NT = T // TB
PAD_ROWS = 1024
XROWS = E * CAP + PAD_ROWS
NC, NS, L = 2, 16, 16
NW = NC * NS


def _router_body(x_ref, gw_ref, dest0_ref, dest1_ref, w0_ref, w1_ref,
                 p1_s, p2_s, i1_s, i2_s):
    i = pl.program_id(0)
    xb = x_ref[...]
    gw = gw_ref[...]
    logits = lax.dot_general(xb, gw, (((1,), (1,)), ((), ())),
                             preferred_element_type=jnp.float32)
    cols = lax.broadcasted_iota(jnp.int32, (TB, E), 1)
    m1 = jnp.max(logits, axis=1, keepdims=True)
    i1 = jnp.min(jnp.where(logits == m1, cols, E), axis=1)
    masked = jnp.where(cols == i1[:, None], -jnp.inf, logits)
    m2 = jnp.max(masked, axis=1, keepdims=True)
    i2 = jnp.min(jnp.where(masked == m2, cols, E), axis=1)
    e2 = jnp.exp(m2 - m1)[:, 0]
    s = 1.0 + e2
    p1 = 1.0 / s
    p2 = e2 / s
    s2 = p1 + p2
    q1 = p1 / s2
    q2 = p2 / s2
    p1_s[i, :] = q1
    p2_s[i, :] = q2
    i1_s[i, :] = i1
    i2_s[i, :] = i2

    @pl.when(i == NT - 1)
    def _select():
        qA = p1_s[...]
        qB = p2_s[...]
        eA = i1_s[...]
        eB = i2_s[...]
        bA = lax.bitcast_convert_type(qA, jnp.int32)
        bB = lax.bitcast_convert_type(qB, jnp.int32)

        capf = jnp.float32(CAP)

        def bs_body(_, carry):
            lo, hi = carry
            nlo, nhi = [], []
            for e in range(E):
                mid = (lo[e] + hi[e]) >> 1
                cnt = (jnp.sum(((eA == e) & (bA > mid)).astype(jnp.float32)) +
                       jnp.sum(((eB == e) & (bB > mid)).astype(jnp.float32)))
                lt = cnt < capf
                nlo.append(jnp.where(lt, lo[e], mid))
                nhi.append(jnp.where(lt, mid, hi[e]))
            return tuple(nlo), tuple(nhi)

        lo0 = tuple(jnp.int32(-1) for _ in range(E))
        hi0 = tuple(jnp.int32(0x3F800001) for _ in range(E))
        _, vstar = lax.fori_loop(0, 31, bs_body, (lo0, hi0))

        rem = []
        for e in range(E):
            cnt_gt = (jnp.sum(((eA == e) & (bA > vstar[e])).astype(jnp.float32)) +
                      jnp.sum(((eB == e) & (bB > vstar[e])).astype(jnp.float32)))
            rem.append(capf - cnt_gt)

        def sel(eplane, per_e):
            acc = jnp.zeros((NT, TB), per_e[0].dtype)
            for e in range(E):
                acc = jnp.where(eplane == e, per_e[e], acc)
            return acc

        vsA = sel(eA, [jnp.broadcast_to(v, (NT, TB)) for v in vstar])
        vsB = sel(eB, [jnp.broadcast_to(v, (NT, TB)) for v in vstar])
        gtA, eqA = bA > vsA, bA == vsA
        gtB, eqB = bB > vsB, bB == vsB

        U = (lax.broadcasted_iota(jnp.int32, (TB, TB), 0) <
             lax.broadcasted_iota(jnp.int32, (TB, TB), 1)).astype(jnp.float32)
        rI = lax.broadcasted_iota(jnp.int32, (2 * E * NT, 2 * E * NT), 0)
        rJ = lax.broadcasted_iota(jnp.int32, (2 * E * NT, 2 * E * NT), 1)
        Loff = (((rI // NT) == (rJ // NT)) & (rJ < rI)).astype(jnp.float32)

        def prefix_planes(planesA, planesB):
            Z = jnp.concatenate(planesA + planesB, axis=0)
            P = lax.dot_general(Z, U, (((1,), (0,)), ((), ())),
                                preferred_element_type=jnp.float32)
            rs = P[:, TB - 1:TB] + Z[:, TB - 1:TB]
            offs = lax.dot_general(Loff, rs, (((1,), (0,)), ((), ())),
                                   preferred_element_type=jnp.float32)
            P = P + offs
            pA = [P[e * NT:(e + 1) * NT, :] for e in range(E)]
            pB = [P[(E + e) * NT:(E + e + 1) * NT, :] for e in range(E)]
            return pA, pB

        eqA_e = [((eA == e) & eqA).astype(jnp.float32) for e in range(E)]
        eqB_e = [((eB == e) & eqB).astype(jnp.float32) for e in range(E)]
        tpA, tpB = prefix_planes(eqA_e, eqB_e)
        tieA = sel(eA, [tpA[e] + tpB[e] for e in range(E)])
        tieB = sel(eB, [tpA[e] + eqA_e[e] + tpB[e] for e in range(E)])
        remA = sel(eA, [jnp.broadcast_to(r, (NT, TB)) for r in rem])
        remB = sel(eB, [jnp.broadcast_to(r, (NT, TB)) for r in rem])
        keepA = gtA | (eqA & (tieA < remA))
        keepB = gtB | (eqB & (tieB < remB))

        kA_e = [((eA == e) & keepA).astype(jnp.float32) for e in range(E)]
        kB_e = [((eB == e) & keepB).astype(jnp.float32) for e in range(E)]
        ppA, ppB = prefix_planes(kA_e, kB_e)
        posA = sel(eA, [ppA[e] + ppB[e] for e in range(E)])
        posB = sel(eB, [ppA[e] + kA_e[e] + ppB[e] for e in range(E)])

        destA = jnp.where(keepA, eA * CAP + posA.astype(jnp.int32),
                          jnp.int32(E * CAP))
        destB = jnp.where(keepB, eB * CAP + posB.astype(jnp.int32),
                          jnp.int32(E * CAP))
        dest0_ref[...] = destA
        dest1_ref[...] = destB
        w0_ref[...] = jnp.where(keepA, qA, 0.0)
        w1_ref[...] = jnp.where(keepB, qB, 0.0)


def _router(x, gate_w):
    return pl.pallas_call(
        _router_body,
        grid=(NT,),
        in_specs=[
            pl.BlockSpec((TB, D_MODEL), lambda i: (i, 0)),
            pl.BlockSpec((E, D_MODEL), lambda i: (0, 0)),
        ],
        out_specs=[
            pl.BlockSpec((NT, TB), lambda i: (0, 0)),
            pl.BlockSpec((NT, TB), lambda i: (0, 0)),
            pl.BlockSpec((NT, TB), lambda i: (0, 0)),
            pl.BlockSpec((NT, TB), lambda i: (0, 0)),
        ],
        out_shape=[
            jax.ShapeDtypeStruct((NT, TB), jnp.int32),
            jax.ShapeDtypeStruct((NT, TB), jnp.int32),
            jax.ShapeDtypeStruct((NT, TB), jnp.float32),
            jax.ShapeDtypeStruct((NT, TB), jnp.float32),
        ],
        scratch_shapes=[
            pltpu.VMEM((NT, TB), jnp.float32),
            pltpu.VMEM((NT, TB), jnp.float32),
            pltpu.VMEM((NT, TB), jnp.int32),
            pltpu.VMEM((NT, TB), jnp.int32),
        ],
        compiler_params=pltpu.CompilerParams(
            dimension_semantics=("arbitrary",)),
    )(x, gate_w)


_SC_MESH = plsc.VectorSubcoreMesh(core_axis_name="c", subcore_axis_name="s")
_SC_PARAMS = pltpu.CompilerParams(needs_layout_passes=False)


def _invert_body(d0_hbm, d1_hbm, w0_hbm, w1_hbm, src_hbm, wts_hbm,
                 d_v, wv_v, src_v, wts_v):
    wid = lax.axis_index("s") * NC + lax.axis_index("c")

    @pl.when(wid == 0)
    def _():
        zi = jnp.zeros((L,), jnp.int32)
        zf = jnp.zeros((L,), jnp.float32)

        def zero_body(i, _):
            src_v[pl.ds(i * L, L)] = zi
            wts_v[pl.ds(i * L, L)] = zf
            return 0

        lax.fori_loop(0, XROWS // L, zero_body, 0)

        lane = lax.iota(jnp.int32, L)

        def scan_plane(d_hbm, w_hbm):
            pltpu.sync_copy(d_hbm, d_v)
            pltpu.sync_copy(w_hbm, wv_v)

            def body(i, _):
                d = d_v[pl.ds(i * L, L)]
                w = wv_v[pl.ds(i * L, L)]
                tok = i * L + lane
                plsc.store_scatter(src_v, [d], tok)
                plsc.store_scatter(wts_v, [d], w)
                return 0

            lax.fori_loop(0, T // L, body, 0)

        scan_plane(d0_hbm, w0_hbm)
        scan_plane(d1_hbm, w1_hbm)
        pltpu.sync_copy(src_v, src_hbm)
        pltpu.sync_copy(wts_v, wts_hbm)


@functools.partial(
    pl.kernel,
    out_type=[jax.ShapeDtypeStruct((XROWS,), jnp.int32),
              jax.ShapeDtypeStruct((XROWS,), jnp.float32)],
    mesh=_SC_MESH,
    scratch_types=[pltpu.VMEM((T,), jnp.int32),
                   pltpu.VMEM((T,), jnp.float32),
                   pltpu.VMEM((XROWS,), jnp.int32),
                   pltpu.VMEM((XROWS,), jnp.float32)],
    compiler_params=_SC_PARAMS,
)
def _invert(*args):
    _invert_body(*args)


RPW = XROWS // NW
GCH = 48
NCH = RPW // GCH


def _gather_body(x_hbm, src_hbm, xe_hbm, idx_v, bufa, bufb, sema, semb):
    wid = lax.axis_index("s") * NC + lax.axis_index("c")
    pltpu.sync_copy(src_hbm.at[wid], idx_v)
    base = wid * RPW
    bufs = (bufa, bufb)
    sems = (sema, semb)
    pltpu.async_copy(x_hbm.at[idx_v.at[0]], bufa, sema)
    for j in range(NCH):
        pltpu.make_async_copy(x_hbm.at[idx_v.at[j]], bufs[j % 2],
                              sems[j % 2]).wait()
        if j + 1 < NCH:
            pltpu.async_copy(x_hbm.at[idx_v.at[j + 1]], bufs[(j + 1) % 2],
                             sems[(j + 1) % 2])
        pltpu.sync_copy(bufs[j % 2], xe_hbm.at[pl.ds(base + j * GCH, GCH)])


@functools.partial(
    pl.kernel,
    out_type=jax.ShapeDtypeStruct((XROWS, D_MODEL), jnp.float32),
    mesh=_SC_MESH,
    scratch_types=[pltpu.VMEM((NCH, GCH), jnp.int32),
                   pltpu.VMEM((GCH, D_MODEL), jnp.float32),
                   pltpu.VMEM((GCH, D_MODEL), jnp.float32),
                   pltpu.SemaphoreType.DMA,
                   pltpu.SemaphoreType.DMA],
    compiler_params=_SC_PARAMS,
)
def _gather(*args):
    _gather_body(*args)


HCH = 1024
NH = (HIDDEN + HCH - 1) // HCH
NE = XROWS // CAP


def _ffn_body(xe_ref, wg_ref, wu_ref, wd_ref, wts_ref, ye_ref, yacc):
    e = pl.program_id(0)
    j = pl.program_id(1)

    @pl.when(jnp.logical_and(e == NE - 1, j == NH - 1))
    def _():
        ye_ref[...] = jnp.zeros((CAP, D_MODEL), jnp.float32)

    @pl.when(e < NE - 1)
    def _():
        _ffn_compute(xe_ref, wg_ref, wu_ref, wd_ref, wts_ref, ye_ref, yacc, j)


def _ffn_compute(xe_ref, wg_ref, wu_ref, wd_ref, wts_ref, ye_ref, yacc, j):
    xb = xe_ref[...].astype(jnp.bfloat16)
    wg = wg_ref[0]
    wu = wu_ref[0]
    wd = wd_ref[0]
    g = lax.dot_general(xb, wg, (((1,), (1,)), ((), ())),
                        preferred_element_type=jnp.float32)
    u = lax.dot_general(xb, wu, (((1,), (1,)), ((), ())),
                        preferred_element_type=jnp.float32)
    hid = lax.broadcasted_iota(jnp.int32, (CAP, HCH), 1) + j * HCH
    hmask = hid < HIDDEN
    h = jnp.where(hmask, g * jax.nn.sigmoid(g) * u, 0.0).astype(jnp.bfloat16)
    wdm = jnp.where(lax.broadcasted_iota(jnp.int32, (HCH, D_MODEL), 0) + j * HCH
                    < HIDDEN, wd, 0).astype(jnp.bfloat16)
    y = lax.dot_general(h, wdm, (((1,), (0,)), ((), ())),
                        preferred_element_type=jnp.float32)

    @pl.when(j == 0)
    def _():
        yacc[...] = y

    @pl.when(j > 0)
    def _():
        yacc[...] = yacc[...] + y

    @pl.when(j == NH - 1)
    def _():
        ye_ref[...] = yacc[...] * wts_ref[0]


def _ffn(xe, w_gate, w_up, w_down, wts3):
    return pl.pallas_call(
        _ffn_body,
        grid=(NE, NH),
        in_specs=[
            pl.BlockSpec((CAP, D_MODEL), lambda e, j: (e, 0)),
            pl.BlockSpec((1, HCH, D_MODEL),
                         lambda e, j: (jnp.minimum(e, E - 1), j, 0)),
            pl.BlockSpec((1, HCH, D_MODEL),
                         lambda e, j: (jnp.minimum(e, E - 1), j, 0)),
            pl.BlockSpec((1, HCH, D_MODEL),
                         lambda e, j: (jnp.minimum(e, E - 1), j, 0)),
            pl.BlockSpec((1, CAP, 1), lambda e, j: (e, 0, 0)),
        ],
        out_specs=pl.BlockSpec((CAP, D_MODEL), lambda e, j: (e, 0)),
        out_shape=jax.ShapeDtypeStruct((XROWS, D_MODEL), jnp.float32),
        scratch_shapes=[pltpu.VMEM((CAP, D_MODEL), jnp.float32)],
        compiler_params=pltpu.CompilerParams(
            dimension_semantics=("arbitrary", "arbitrary")),
    )(xe, w_gate, w_up, w_down, wts3)


TPW = T // NW
CCH = 16
NCC = TPW // CCH


def _combine_body(ye_hbm, d01_hbm, out_hbm, i_v,
                  ba, bb, oba, obb, sema, semb, semoa, semob):
    wid = lax.axis_index("s") * NC + lax.axis_index("c")
    wbase = wid * TPW
    pltpu.sync_copy(d01_hbm.at[wid], i_v)
    bs, obs = (ba, bb), (oba, obb)
    sems = (sema, semb)
    semos = (semoa, semob)

    def issue(c, p):
        pltpu.async_copy(ye_hbm.at[i_v.at[c]], bs[p], sems[p])

    def wait(c, p):
        pltpu.make_async_copy(ye_hbm.at[i_v.at[c]], bs[p], sems[p]).wait()

    def out_wait(p):
        pltpu.make_async_copy(obs[p], out_hbm.at[pl.ds(wbase, CCH)],
                              semos[p]).wait()

    def compute_store(c, p):
        b, ob = bs[p], obs[p]

        @pl.when(c >= 2)
        def _():
            out_wait(p)

        def row_body(r, _):
            for k in range(D_MODEL // L):
                ob[r, pl.ds(k * L, L)] = (b[r, pl.ds(k * L, L)] +
                                          b[CCH + r, pl.ds(k * L, L)])
            return 0

        lax.fori_loop(0, CCH, row_body, 0)
        pltpu.async_copy(ob, out_hbm.at[pl.ds(wbase + c * CCH, CCH)], semos[p])

    issue(0, 0)

    def step(s, _):
        c0 = s * 2
        wait(c0, 0)
        issue(c0 + 1, 1)
        compute_store(c0, 0)
        wait(c0 + 1, 1)

        @pl.when(s < NCC // 2 - 1)
        def _():
            issue(c0 + 2, 0)

        compute_store(c0 + 1, 1)
        return 0

    lax.fori_loop(0, NCC // 2, step, 0)
    out_wait(0)
    out_wait(1)


@functools.partial(
    pl.kernel,
    out_type=jax.ShapeDtypeStruct((T, D_MODEL), jnp.float32),
    mesh=_SC_MESH,
    scratch_types=[pltpu.VMEM((NCC, 2 * CCH), jnp.int32),
                   pltpu.VMEM((2 * CCH, D_MODEL), jnp.float32),
                   pltpu.VMEM((2 * CCH, D_MODEL), jnp.float32),
                   pltpu.VMEM((CCH, D_MODEL), jnp.float32),
                   pltpu.VMEM((CCH, D_MODEL), jnp.float32),
                   pltpu.SemaphoreType.DMA,
                   pltpu.SemaphoreType.DMA,
                   pltpu.SemaphoreType.DMA,
                   pltpu.SemaphoreType.DMA],
    compiler_params=_SC_PARAMS,
)
def _combine(*args):
    _combine_body(*args)


def kernel(x, gate_w, w_gate, w_up, w_down):
    dest0, dest1, w0, w1 = _router(x, gate_w)
    src, wts = _invert(dest0.reshape(T), dest1.reshape(T),
                       w0.reshape(T), w1.reshape(T))
    xe = _gather(x, src.reshape(NW, NCH, GCH))
    wgb = w_gate.astype(jnp.bfloat16)
    wub = w_up.astype(jnp.bfloat16)
    wdb = w_down.transpose(0, 2, 1).astype(jnp.bfloat16)
    ye = _ffn(xe, wgb, wub, wdb, wts.reshape(NE, CAP, 1))
    d01 = jnp.stack([dest0.reshape(NW, NCC, CCH),
                     dest1.reshape(NW, NCC, CCH)],
                    axis=2).reshape(NW, NCC, 2 * CCH)
    out = _combine(ye, d01)
    return out
